# Initial kernel scaffold; baseline (speedup 1.0000x reference)
import jax
import jax.numpy as jnp
from jax.experimental import pallas as pl


def kernel(x, edge_index, edge_type, W_in, b_in, Wr, br, W1, b1, W2, b2, Wp, bp):
    raise NotImplementedError("write your pallas kernel here")


# speedup vs baseline: 8.8253x; 8.8253x over previous
import jax
import jax.numpy as jnp
from jax import lax
from jax.experimental import pallas as pl
from jax.experimental.pallas import tpu as pltpu
from jax.experimental.pallas import tpu_sc as plsc

_N = 10000
_E = 320000
_D = 128
_R = 6
_NP = 10240
_NC = 2
_NS = 16
_DH = _D // _NC
_EP = 327680
_CH = 80
_NJ = _EP // (_NS * _CH)
_RPT = _NP // _NS
_BLK = 512
_NBLK = _NP // _BLK


def _table_body(x_ref, win_ref, bin_ref, wr_ref, br_ref, h_ref, t_ref):
    xb = x_ref[...]
    h = jnp.maximum(
        lax.dot_general(xb, win_ref[...], (((1,), (1,)), ((), ())),
                        preferred_element_type=jnp.float32) + bin_ref[...],
        0.0)
    h_ref[...] = h
    for r in range(_R):
        row = lax.dot_general(h, wr_ref[r], (((1,), (1,)), ((), ())),
                              preferred_element_type=jnp.float32) + br_ref[r]
        for c in range(_NC):
            t_ref[c, r] = row[:, c * _DH:(c + 1) * _DH]


def _table_call(x_p, W_in, b_in, Wr, br):
    return pl.pallas_call(
        _table_body,
        grid=(_NBLK,),
        in_specs=[
            pl.BlockSpec((_BLK, _D), lambda i: (i, 0)),
            pl.BlockSpec((_D, _D), lambda i: (0, 0)),
            pl.BlockSpec((1, _D), lambda i: (0, 0)),
            pl.BlockSpec((_R, _D, _D), lambda i: (0, 0, 0)),
            pl.BlockSpec((_R, 1, _D), lambda i: (0, 0, 0)),
        ],
        out_specs=[
            pl.BlockSpec((_BLK, _D), lambda i: (i, 0)),
            pl.BlockSpec((_NC, _R, _BLK, _DH), lambda i: (0, 0, i, 0)),
        ],
        out_shape=[
            jax.ShapeDtypeStruct((_NP, _D), jnp.float32),
            jax.ShapeDtypeStruct((_NC, _R, _NP, _DH), jnp.float32),
        ],
    )(x_p, W_in, b_in, Wr, br)


def _gidx_body(src_ref, et_ref, gidx_ref):
    gidx_ref[...] = et_ref[...] * _NP + src_ref[...]


def _gidx_call(src2d, et2d):
    return pl.pallas_call(
        _gidx_body,
        out_shape=jax.ShapeDtypeStruct(src2d.shape, jnp.int32),
    )(src2d, et2d)


def _sc_body(t0_hbm, t1_hbm, gidx_hbm, dst_hbm, ones_hbm, zrow_hbm, zcnt_hbm,
             msgp_hbm, cntp_hbm,
             idx_v, dst_v, rows_v, ones_v, msg_sh, cnt_sh, gsem):
    c = lax.axis_index("c")
    s = lax.axis_index("s")

    for k in range(_RPT // _CH):
        pltpu.sync_copy(zrow_hbm, msg_sh.at[pl.ds(s * _RPT + k * _CH, _CH)])
        pltpu.sync_copy(zcnt_hbm, cnt_sh.at[pl.ds(s * _RPT + k * _CH, _CH)])
    pltpu.sync_copy(ones_hbm, ones_v)
    pltpu.sync_copy(gidx_hbm.at[s], idx_v)
    pltpu.sync_copy(dst_hbm.at[s], dst_v)
    plsc.subcore_barrier()

    def step(j, carry):
        @pl.when(c == 0)
        def _():
            pltpu.async_copy(t0_hbm.at[idx_v.at[j]], rows_v, gsem).wait()

        @pl.when(c == 1)
        def _():
            pltpu.async_copy(t1_hbm.at[idx_v.at[j]], rows_v, gsem).wait()

        pltpu.sync_copy(rows_v, msg_sh.at[dst_v.at[j]], add=True)
        pltpu.sync_copy(ones_v, cnt_sh.at[dst_v.at[j]], add=True)
        return carry

    lax.fori_loop(0, _NJ, step, 0)

    plsc.subcore_barrier()
    pltpu.sync_copy(msg_sh.at[pl.ds(s * _RPT, _RPT)],
                    msgp_hbm.at[c, pl.ds(s * _RPT, _RPT)])
    pltpu.sync_copy(cnt_sh.at[pl.ds(s * _RPT, _RPT)],
                    cntp_hbm.at[c, pl.ds(s * _RPT, _RPT)])


def _sc_call(t0, t1, gidx, dst, ones, zrow, zcnt):
    fn = pl.kernel(
        _sc_body,
        out_type=[
            jax.ShapeDtypeStruct((_NC, _NP, _DH), jnp.float32),
            jax.ShapeDtypeStruct((_NC, _NP, 16), jnp.float32),
        ],
        mesh=plsc.VectorSubcoreMesh(core_axis_name="c", subcore_axis_name="s"),
        compiler_params=pltpu.CompilerParams(use_tc_tiling_on_sc=False),
        scratch_types=[
            pltpu.VMEM((_NJ, _CH), jnp.int32),
            pltpu.VMEM((_NJ, _CH), jnp.int32),
            pltpu.VMEM((_CH, _DH), jnp.float32),
            pltpu.VMEM((_CH, 16), jnp.float32),
            pltpu.VMEM_SHARED((_NP, _DH), jnp.float32),
            pltpu.VMEM_SHARED((_NP, 16), jnp.float32),
            pltpu.SemaphoreType.DMA,
        ],
    )
    return fn(t0, t1, gidx, dst, ones, zrow, zcnt)


def _epi_body(h_ref, msgp_ref, cntp_ref, w1_ref, b1_ref, w2_ref, b2_ref,
              wp_ref, emb_ref, pred_ref):
    msg = jnp.concatenate([msgp_ref[0], msgp_ref[1]], axis=1)
    cnt = cntp_ref[0, :, 0:1]
    t = h_ref[...] + msg / jnp.maximum(cnt, 1.0)
    z = jnp.maximum(
        lax.dot_general(t, w1_ref[...], (((1,), (1,)), ((), ())),
                        preferred_element_type=jnp.float32) + b1_ref[...],
        0.0)
    emb = lax.dot_general(z, w2_ref[...], (((1,), (1,)), ((), ())),
                          preferred_element_type=jnp.float32) + b2_ref[...]
    emb_ref[...] = emb
    pred_ref[...] = lax.dot_general(emb, wp_ref[...], (((1,), (1,)), ((), ())),
                                    preferred_element_type=jnp.float32)


def _epi_call(h, msgp, cntp, W1, b1, W2, b2, Wp):
    return pl.pallas_call(
        _epi_body,
        grid=(_NBLK,),
        in_specs=[
            pl.BlockSpec((_BLK, _D), lambda i: (i, 0)),
            pl.BlockSpec((_NC, _BLK, _DH), lambda i: (0, i, 0)),
            pl.BlockSpec((_NC, _BLK, 16), lambda i: (0, i, 0)),
            pl.BlockSpec((_D, _D), lambda i: (0, 0)),
            pl.BlockSpec((1, _D), lambda i: (0, 0)),
            pl.BlockSpec((_D, _D), lambda i: (0, 0)),
            pl.BlockSpec((1, _D), lambda i: (0, 0)),
            pl.BlockSpec((1, _D), lambda i: (0, 0)),
        ],
        out_specs=[
            pl.BlockSpec((_BLK, _D), lambda i: (i, 0)),
            pl.BlockSpec((_BLK, 1), lambda i: (i, 0)),
        ],
        out_shape=[
            jax.ShapeDtypeStruct((_NP, _D), jnp.float32),
            jax.ShapeDtypeStruct((_NP, 1), jnp.float32),
        ],
    )(h, msgp, cntp, W1, b1, W2, b2, Wp)


@jax.jit
def kernel(x, edge_index, edge_type, W_in, b_in, Wr, br, W1, b1, W2, b2, Wp, bp):
    x_p = jnp.pad(x, ((0, _NP - _N), (0, 0)))
    h, T = _table_call(x_p, W_in, b_in.reshape(1, _D), Wr, br.reshape(_R, 1, _D))
    t0 = T[0].reshape(_R * _NP, _DH)
    t1 = T[1].reshape(_R * _NP, _DH)

    pad = _EP - _E
    src_p = jnp.pad(edge_index[0], (0, pad))
    et_p = jnp.pad(edge_type, (0, pad))
    dst_p = jnp.pad(edge_index[1], (0, pad), constant_values=_N)
    gidx = _gidx_call(src_p.reshape(_EP // _D, _D),
                      et_p.reshape(_EP // _D, _D)).reshape(_NS, _NJ, _CH)
    dst = dst_p.reshape(_NS, _NJ, _CH)

    ones = jnp.ones((_CH, 16), jnp.float32)
    zrow = jnp.zeros((_CH, _DH), jnp.float32)
    zcnt = jnp.zeros((_CH, 16), jnp.float32)
    msgp, cntp = _sc_call(t0, t1, gidx, dst, ones, zrow, zcnt)

    emb_p, pred_p = _epi_call(h, msgp, cntp, W1, b1.reshape(1, _D), W2,
                              b2.reshape(1, _D), Wp)
    return emb_p[:_N], pred_p[:_N] + bp


# speedup vs baseline: 12.4189x; 1.4072x over previous
import jax
import jax.numpy as jnp
from jax import lax
from jax.experimental import pallas as pl
from jax.experimental.pallas import tpu as pltpu
from jax.experimental.pallas import tpu_sc as plsc

_N = 10000
_E = 320000
_D = 128
_R = 6
_NP = 10240
_NC = 2
_NS = 16
_DH = _D // _NC
_EP = 327680
_CH = 128
_NJ = _EP // (_NS * _CH)
_RPT = _NP // _NS
_BLK = 512
_NBLK = _NP // _BLK


def _table_body(x_ref, win_ref, bin_ref, wr_ref, br_ref, src_ref, et_ref,
                h_ref, t_ref, gidx_ref):
    gidx_ref[...] = et_ref[...] * _NP + src_ref[...]
    xb = x_ref[...]
    h = jnp.maximum(
        lax.dot_general(xb, win_ref[...], (((1,), (1,)), ((), ())),
                        preferred_element_type=jnp.float32) + bin_ref[...],
        0.0)
    h_ref[...] = h
    for r in range(_R):
        row = lax.dot_general(h, wr_ref[r], (((1,), (1,)), ((), ())),
                              preferred_element_type=jnp.float32) + br_ref[r]
        for c in range(_NC):
            t_ref[c, r] = row[:, c * _DH:(c + 1) * _DH]


def _table_call(x_p, W_in, b_in, Wr, br, src2d, et2d):
    return pl.pallas_call(
        _table_body,
        grid=(_NBLK,),
        in_specs=[
            pl.BlockSpec((_BLK, _D), lambda i: (i, 0)),
            pl.BlockSpec((_D, _D), lambda i: (0, 0)),
            pl.BlockSpec((1, _D), lambda i: (0, 0)),
            pl.BlockSpec((_R, _D, _D), lambda i: (0, 0, 0)),
            pl.BlockSpec((_R, 1, _D), lambda i: (0, 0, 0)),
            pl.BlockSpec((_EP // _D // _NBLK, _D), lambda i: (i, 0)),
            pl.BlockSpec((_EP // _D // _NBLK, _D), lambda i: (i, 0)),
        ],
        out_specs=[
            pl.BlockSpec((_BLK, _D), lambda i: (i, 0)),
            pl.BlockSpec((_NC, _R, _BLK, _DH), lambda i: (0, 0, i, 0)),
            pl.BlockSpec((_EP // _D // _NBLK, _D), lambda i: (i, 0)),
        ],
        out_shape=[
            jax.ShapeDtypeStruct((_NP, _D), jnp.float32),
            jax.ShapeDtypeStruct((_NC, _R, _NP, _DH), jnp.float32),
            jax.ShapeDtypeStruct((_EP // _D, _D), jnp.int32),
        ],
    )(x_p, W_in, b_in, Wr, br, src2d, et2d)


def _sc_body(t0_hbm, t1_hbm, gidx_hbm, dst_hbm, ones_hbm, zrow_hbm, zcnt_hbm,
             msgp_hbm, cntp_hbm,
             idx_v, dst_v, rows_a, rows_b, ones_v, msg_sh, cnt_sh,
             gsem_a, gsem_b):
    c = lax.axis_index("c")
    s = lax.axis_index("s")

    for k in range(_RPT // _CH):
        pltpu.sync_copy(zrow_hbm, msg_sh.at[pl.ds(s * _RPT + k * _CH, _CH)])
        pltpu.sync_copy(zcnt_hbm, cnt_sh.at[pl.ds(s * _RPT + k * _CH, _CH)])
    pltpu.sync_copy(ones_hbm, ones_v)
    pltpu.sync_copy(gidx_hbm.at[s], idx_v)
    pltpu.sync_copy(dst_hbm.at[s], dst_v)
    plsc.subcore_barrier()

    def start(j, buf, sem):
        @pl.when(c == 0)
        def _():
            pltpu.async_copy(t0_hbm.at[idx_v.at[j]], buf, sem)

        @pl.when(c == 1)
        def _():
            pltpu.async_copy(t1_hbm.at[idx_v.at[j]], buf, sem)

    def wait(buf, sem):
        pltpu.make_async_copy(t0_hbm.at[pl.ds(0, _CH)], buf, sem).wait()

    start(0, rows_a, gsem_a)
    start(1, rows_b, gsem_b)

    def step(i, carry):
        j = i * 2
        wait(rows_a, gsem_a)
        pltpu.sync_copy(rows_a, msg_sh.at[dst_v.at[j]], add=True)
        pltpu.sync_copy(ones_v, cnt_sh.at[dst_v.at[j]], add=True)

        @pl.when(j + 2 < _NJ)
        def _():
            start(j + 2, rows_a, gsem_a)

        wait(rows_b, gsem_b)
        pltpu.sync_copy(rows_b, msg_sh.at[dst_v.at[j + 1]], add=True)
        pltpu.sync_copy(ones_v, cnt_sh.at[dst_v.at[j + 1]], add=True)

        @pl.when(j + 3 < _NJ)
        def _():
            start(j + 3, rows_b, gsem_b)

        return carry

    lax.fori_loop(0, _NJ // 2, step, 0)

    plsc.subcore_barrier()
    pltpu.sync_copy(msg_sh.at[pl.ds(s * _RPT, _RPT)],
                    msgp_hbm.at[c, pl.ds(s * _RPT, _RPT)])
    pltpu.sync_copy(cnt_sh.at[pl.ds(s * _RPT, _RPT)],
                    cntp_hbm.at[c, pl.ds(s * _RPT, _RPT)])


def _sc_call(t0, t1, gidx, dst, ones, zrow, zcnt):
    fn = pl.kernel(
        _sc_body,
        out_type=[
            jax.ShapeDtypeStruct((_NC, _NP, _DH), jnp.float32),
            jax.ShapeDtypeStruct((_NC, _NP, 16), jnp.float32),
        ],
        mesh=plsc.VectorSubcoreMesh(core_axis_name="c", subcore_axis_name="s"),
        compiler_params=pltpu.CompilerParams(use_tc_tiling_on_sc=False),
        scratch_types=[
            pltpu.VMEM((_NJ, _CH), jnp.int32),
            pltpu.VMEM((_NJ, _CH), jnp.int32),
            pltpu.VMEM((_CH, _DH), jnp.float32),
            pltpu.VMEM((_CH, _DH), jnp.float32),
            pltpu.VMEM((_CH, 16), jnp.float32),
            pltpu.VMEM_SHARED((_NP, _DH), jnp.float32),
            pltpu.VMEM_SHARED((_NP, 16), jnp.float32),
            pltpu.SemaphoreType.DMA,
            pltpu.SemaphoreType.DMA,
        ],
    )
    return fn(t0, t1, gidx, dst, ones, zrow, zcnt)


def _epi_body(h_ref, msgp_ref, cntp_ref, w1_ref, b1_ref, w2_ref, b2_ref,
              wp_ref, emb_ref, pred_ref):
    msg = jnp.concatenate([msgp_ref[0], msgp_ref[1]], axis=1)
    cnt = cntp_ref[0, :, 0:1]
    t = h_ref[...] + msg / jnp.maximum(cnt, 1.0)
    z = jnp.maximum(
        lax.dot_general(t, w1_ref[...], (((1,), (1,)), ((), ())),
                        preferred_element_type=jnp.float32) + b1_ref[...],
        0.0)
    emb = lax.dot_general(z, w2_ref[...], (((1,), (1,)), ((), ())),
                          preferred_element_type=jnp.float32) + b2_ref[...]
    emb_ref[...] = emb
    pred_ref[...] = lax.dot_general(emb, wp_ref[...], (((1,), (1,)), ((), ())),
                                    preferred_element_type=jnp.float32)


def _epi_call(h, msgp, cntp, W1, b1, W2, b2, Wp):
    return pl.pallas_call(
        _epi_body,
        grid=(_NBLK,),
        in_specs=[
            pl.BlockSpec((_BLK, _D), lambda i: (i, 0)),
            pl.BlockSpec((_NC, _BLK, _DH), lambda i: (0, i, 0)),
            pl.BlockSpec((_NC, _BLK, 16), lambda i: (0, i, 0)),
            pl.BlockSpec((_D, _D), lambda i: (0, 0)),
            pl.BlockSpec((1, _D), lambda i: (0, 0)),
            pl.BlockSpec((_D, _D), lambda i: (0, 0)),
            pl.BlockSpec((1, _D), lambda i: (0, 0)),
            pl.BlockSpec((1, _D), lambda i: (0, 0)),
        ],
        out_specs=[
            pl.BlockSpec((_BLK, _D), lambda i: (i, 0)),
            pl.BlockSpec((_BLK, 1), lambda i: (i, 0)),
        ],
        out_shape=[
            jax.ShapeDtypeStruct((_NP, _D), jnp.float32),
            jax.ShapeDtypeStruct((_NP, 1), jnp.float32),
        ],
    )(h, msgp, cntp, W1, b1, W2, b2, Wp)


@jax.jit
def kernel(x, edge_index, edge_type, W_in, b_in, Wr, br, W1, b1, W2, b2, Wp, bp):
    x_p = jnp.pad(x, ((0, _NP - _N), (0, 0)))
    pad = _EP - _E
    src_p = jnp.pad(edge_index[0], (0, pad))
    et_p = jnp.pad(edge_type, (0, pad))
    dst_p = jnp.pad(edge_index[1], (0, pad), constant_values=_N)
    h, T, gidx2d = _table_call(x_p, W_in, b_in.reshape(1, _D), Wr,
                               br.reshape(_R, 1, _D),
                               src_p.reshape(_EP // _D, _D),
                               et_p.reshape(_EP // _D, _D))
    t0 = T[0].reshape(_R * _NP, _DH)
    t1 = T[1].reshape(_R * _NP, _DH)
    gidx = gidx2d.reshape(_NS, _NJ, _CH)
    dst = dst_p.reshape(_NS, _NJ, _CH)

    ones = jnp.ones((_CH, 16), jnp.float32)
    zrow = jnp.zeros((_CH, _DH), jnp.float32)
    zcnt = jnp.zeros((_CH, 16), jnp.float32)
    msgp, cntp = _sc_call(t0, t1, gidx, dst, ones, zrow, zcnt)

    emb_p, pred_p = _epi_call(h, msgp, cntp, W1, b1.reshape(1, _D), W2,
                              b2.reshape(1, _D), Wp)
    return emb_p[:_N], pred_p[:_N] + bp
